# Initial kernel scaffold; baseline (speedup 1.0000x reference)
import jax
import jax.numpy as jnp
from jax.experimental import pallas as pl


def kernel(x, edge_index, pair_idxs_left, pair_idxs_right, y, W1, b1, Wg, bg, W2, b2, W3, b3):
    raise NotImplementedError("write your pallas kernel here")


# speedup vs baseline: 7.9823x; 7.9823x over previous
import functools

import jax
import jax.numpy as jnp
from jax import lax
from jax.experimental import pallas as pl
from jax.experimental.pallas import tpu as pltpu
from jax.experimental.pallas import tpu_sc as plsc

NC = 2
NS = 16
L = 128


def _stage_a(x, W1, b1, Wg, rb):
    n, d = x.shape
    h = W1.shape[1]

    def body(x_ref, w1_ref, b1_ref, wg_ref, out_ref):
        h1 = jnp.maximum(
            jnp.dot(x_ref[...], w1_ref[...],
                    preferred_element_type=jnp.float32) + b1_ref[...][None, :],
            0.0)
        out_ref[...] = jnp.dot(h1, wg_ref[...],
                               preferred_element_type=jnp.float32)

    return pl.pallas_call(
        body,
        grid=(n // rb,),
        in_specs=[
            pl.BlockSpec((rb, d), lambda i: (i, 0)),
            pl.BlockSpec((d, h), lambda i: (0, 0)),
            pl.BlockSpec((h,), lambda i: (0,)),
            pl.BlockSpec((h, h), lambda i: (0, 0)),
        ],
        out_specs=pl.BlockSpec((rb, h), lambda i: (i, 0)),
        out_shape=jax.ShapeDtypeStruct((n, h), jnp.float32),
    )(x, W1, b1, Wg)


def _stage_b(hw, c0, c1, rb):
    n, h = hw.shape
    hh = h // 2

    def body(hw_ref, c0_ref, c1_ref, hws0_ref, hws1_ref, dinv_ref):
        deg = c0_ref[...] + c1_ref[...] + 1.0
        dinv = lax.rsqrt(deg)
        hws = hw_ref[...] * dinv
        hws0_ref[...] = hws[:, :hh]
        hws1_ref[...] = hws[:, hh:]
        dinv_ref[...] = dinv

    return pl.pallas_call(
        body,
        grid=(n // rb,),
        in_specs=[
            pl.BlockSpec((rb, h), lambda i: (i, 0)),
            pl.BlockSpec((rb, 1), lambda i: (i, 0)),
            pl.BlockSpec((rb, 1), lambda i: (i, 0)),
        ],
        out_specs=(
            pl.BlockSpec((rb, hh), lambda i: (i, 0)),
            pl.BlockSpec((rb, hh), lambda i: (i, 0)),
            pl.BlockSpec((rb, 1), lambda i: (i, 0)),
        ),
        out_shape=(
            jax.ShapeDtypeStruct((n, hh), jnp.float32),
            jax.ShapeDtypeStruct((n, hh), jnp.float32),
            jax.ShapeDtypeStruct((n, 1), jnp.float32),
        ),
    )(hw, c0, c1)


def _stage_c(agg0, agg1, hw, dinv, bg, W2, b2, W3, b3, rb):
    n, h = hw.shape
    h2 = W2.shape[1]
    o = W3.shape[1]
    hh = h // 2

    def body(a0_ref, a1_ref, hw_ref, dv_ref, bg_ref, w2_ref, b2_ref,
             w3_ref, b3_ref, out_ref):
        agg_pre = jnp.concatenate([a0_ref[...], a1_ref[...]], axis=1)
        dv = dv_ref[...]
        hnode = jnp.maximum(
            dv * agg_pre + (dv * dv) * hw_ref[...] + bg_ref[...][None, :], 0.0)
        t = jnp.maximum(
            jnp.dot(hnode, w2_ref[...],
                    preferred_element_type=jnp.float32) + b2_ref[...][None, :],
            0.0)
        out_ref[...] = jnp.dot(
            t, w3_ref[...], preferred_element_type=jnp.float32) \
            + b3_ref[...][None, :]

    return pl.pallas_call(
        body,
        grid=(n // rb,),
        in_specs=[
            pl.BlockSpec((rb, hh), lambda i: (i, 0)),
            pl.BlockSpec((rb, hh), lambda i: (i, 0)),
            pl.BlockSpec((rb, h), lambda i: (i, 0)),
            pl.BlockSpec((rb, 1), lambda i: (i, 0)),
            pl.BlockSpec((h,), lambda i: (0,)),
            pl.BlockSpec((h, h2), lambda i: (0, 0)),
            pl.BlockSpec((h2,), lambda i: (0,)),
            pl.BlockSpec((h2, o), lambda i: (0, 0)),
            pl.BlockSpec((o,), lambda i: (0,)),
        ],
        out_specs=pl.BlockSpec((rb, o), lambda i: (i, 0)),
        out_shape=jax.ShapeDtypeStruct((n, o), jnp.float32),
    )(agg0, agg1, hw, dinv, bg, W2, b2, W3, b3)


def _make_deg_kernel(n_rows, n_pad):
    rpw = n_rows // (NC * NS)
    mesh = plsc.VectorSubcoreMesh(core_axis_name="c", subcore_axis_name="s")

    @functools.partial(
        pl.kernel,
        out_type=jax.ShapeDtypeStruct((NC, n_pad), jnp.float32),
        mesh=mesh,
        scratch_types=[
            pltpu.VMEM((rpw, L), jnp.int32),
            pltpu.VMEM((L,), jnp.float32),
            pltpu.VMEM_SHARED((n_pad,), jnp.float32),
            pltpu.SemaphoreType.DMA,
        ],
    )
    def deg_kernel(dst2_hbm, ones_hbm, zeros1_hbm, out_hbm,
                   idx_v, ones_v, hist_sh, sem):
        c = lax.axis_index("c")
        s = lax.axis_index("s")
        w = c * NS + s

        @pl.when(s == 0)
        def _():
            pltpu.sync_copy(zeros1_hbm, hist_sh)
        pltpu.sync_copy(ones_hbm, ones_v)
        pltpu.sync_copy(dst2_hbm.at[pl.ds(w * rpw, rpw)], idx_v)
        plsc.subcore_barrier()

        descs = []
        for t in range(rpw):
            descs.append(pltpu.async_copy(
                ones_v, hist_sh.at[idx_v.at[t]], sem, add=True))
        for d in descs:
            d.wait()

        plsc.subcore_barrier()

        @pl.when(s == 0)
        def _():
            pltpu.sync_copy(hist_sh, out_hbm.at[c])

    return deg_kernel


def _make_agg_kernel(n_rows, n, n_pad, hh):
    rpw = n_rows // NS
    bat = 2
    nb = rpw // bat
    wo_w = NS
    while n % wo_w != 0 or (n // wo_w) % 8 != 0:
        wo_w -= 1
    nps = n // wo_w
    mesh = plsc.VectorSubcoreMesh(core_axis_name="c", subcore_axis_name="s")

    @functools.partial(
        pl.kernel,
        out_type=(jax.ShapeDtypeStruct((n, hh), jnp.float32),
                  jax.ShapeDtypeStruct((n, hh), jnp.float32)),
        mesh=mesh,
        scratch_types=[
            pltpu.VMEM((bat, L), jnp.int32),
            pltpu.VMEM((bat, L), jnp.int32),
            pltpu.VMEM((bat * L, hh), jnp.float32),
            pltpu.VMEM_SHARED((n_pad, hh), jnp.float32),
            pltpu.SemaphoreType.DMA,
            pltpu.SemaphoreType.DMA,
        ],
    )
    def agg_kernel(hws0_hbm, hws1_hbm, src2_hbm, dst2_hbm, zeros2_hbm,
                   out0_hbm, out1_hbm,
                   idxs_v, idxd_v, rows_v, agg_sh, gsem, ssem):
        c = lax.axis_index("c")
        s = lax.axis_index("s")

        @pl.when(s == 0)
        def _():
            pltpu.sync_copy(zeros2_hbm, agg_sh)
        plsc.subcore_barrier()

        def run(tbl_hbm):
            for b in range(nb):
                row0 = s * rpw + b * bat
                pltpu.sync_copy(src2_hbm.at[pl.ds(row0, bat)], idxs_v)
                pltpu.sync_copy(dst2_hbm.at[pl.ds(row0, bat)], idxd_v)
                gds = []
                for j in range(bat):
                    gds.append(pltpu.async_copy(
                        tbl_hbm.at[idxs_v.at[j]],
                        rows_v.at[pl.ds(j * L, L)], gsem))
                for d in gds:
                    d.wait()
                sds = []
                for j in range(bat):
                    sds.append(pltpu.async_copy(
                        rows_v.at[pl.ds(j * L, L)],
                        agg_sh.at[idxd_v.at[j]], ssem, add=True))
                for d in sds:
                    d.wait()

        @pl.when(c == 0)
        def _():
            run(hws0_hbm)

        @pl.when(c == 1)
        def _():
            run(hws1_hbm)

        plsc.subcore_barrier()

        @pl.when(jnp.logical_and(c == 0, s < wo_w))
        def _():
            pltpu.sync_copy(agg_sh.at[pl.ds(s * nps, nps)],
                            out0_hbm.at[pl.ds(s * nps, nps)])

        @pl.when(jnp.logical_and(c == 1, s < wo_w))
        def _():
            pltpu.sync_copy(agg_sh.at[pl.ds(s * nps, nps)],
                            out1_hbm.at[pl.ds(s * nps, nps)])

    return agg_kernel


def _make_pair_kernel(n_idx_rows, p, o, left_rows):
    bat = 4
    n_batches = n_idx_rows // bat
    tail_valid = p - (left_rows - 1) * L
    mesh = plsc.VectorSubcoreMesh(core_axis_name="c", subcore_axis_name="s")

    @functools.partial(
        pl.kernel,
        out_type=jax.ShapeDtypeStruct((2 * p, o), jnp.float32),
        mesh=mesh,
        scratch_types=[
            pltpu.VMEM((bat, L), jnp.int32),
            pltpu.VMEM((bat * L, o), jnp.float32),
            pltpu.SemaphoreType.DMA,
        ],
    )
    def pair_kernel(etab_hbm, idx2_hbm, out_hbm, idx_v, rows_v, sem):
        c = lax.axis_index("c")
        s = lax.axis_index("s")
        w = c * NS + s
        nw = NC * NS
        nt = (n_batches + nw - 1) // nw

        for t in range(nt):
            b = w + t * nw

            @pl.when(b < n_batches)
            def _():
                pltpu.sync_copy(idx2_hbm.at[pl.ds(b * bat, bat)], idx_v)
                gds = []
                for j in range(bat):
                    gds.append(pltpu.async_copy(
                        etab_hbm.at[idx_v.at[j]],
                        rows_v.at[pl.ds(j * L, L)], sem))
                for d in gds:
                    d.wait()
                for j in range(bat):
                    k = b * bat + j
                    base = jnp.where(k < left_rows,
                                     k * L,
                                     p + (k - left_rows) * L)
                    is_tail = jnp.logical_or(k == left_rows - 1,
                                             k == 2 * left_rows - 1)

                    @pl.when(jnp.logical_not(is_tail))
                    def _():
                        pltpu.sync_copy(rows_v.at[pl.ds(j * L, L)],
                                        out_hbm.at[pl.ds(base, L)])

                    @pl.when(is_tail)
                    def _():
                        pltpu.sync_copy(
                            rows_v.at[pl.ds(j * L, tail_valid)],
                            out_hbm.at[pl.ds(base, tail_valid)])

    return pair_kernel


def kernel(x, edge_index, pair_idxs_left, pair_idxs_right, y,
           W1, b1, Wg, bg, W2, b2, W3, b3):
    n, d = x.shape
    e = edge_index.shape[1]
    p = pair_idxs_left.shape[0]
    h = W1.shape[1]
    o = W3.shape[1]
    hh = h // 2
    rb = 1000 if n % 1000 == 0 else 8

    unit = NC * NS * L
    e_pad = -(-e // unit) * unit
    src_p = jnp.concatenate(
        [edge_index[0], jnp.zeros((e_pad - e,), jnp.int32)]).reshape(-1, L)
    dst_p = jnp.concatenate(
        [edge_index[1], jnp.full((e_pad - e,), n, jnp.int32)]).reshape(-1, L)
    n_rows = e_pad // L
    n_pad1 = -(-(n + 1) // L) * L
    n_pad2 = n + 8

    ones_l = jnp.ones((L,), jnp.float32)
    zeros1 = jnp.zeros((n_pad1,), jnp.float32)
    zeros2 = jnp.zeros((n_pad2, hh), jnp.float32)

    counts = _make_deg_kernel(n_rows, n_pad1)(dst_p, ones_l, zeros1)
    hw = _stage_a(x, W1, b1, Wg, rb)
    c0 = counts[0, :n][:, None]
    c1 = counts[1, :n][:, None]
    hws0, hws1, dinv = _stage_b(hw, c0, c1, rb)
    agg0, agg1 = _make_agg_kernel(n_rows, n, n_pad2, hh)(
        hws0, hws1, src_p, dst_p, zeros2)
    etab = _stage_c(agg0, agg1, hw, dinv, bg, W2, b2, W3, b3, rb)

    left_rows = -(-p // L)
    ipad = left_rows * L - p
    zpad = jnp.zeros((ipad,), jnp.int32)
    idx_all = jnp.concatenate(
        [pair_idxs_left, zpad, pair_idxs_right, zpad]).reshape(-1, L)
    n_idx_rows = idx_all.shape[0]
    flat = _make_pair_kernel(n_idx_rows, p, o, left_rows)(etab, idx_all)
    return flat.reshape(2, p, o), y


# speedup vs baseline: 9.4462x; 1.1834x over previous
import functools

import jax
import jax.numpy as jnp
from jax import lax
from jax.experimental import pallas as pl
from jax.experimental.pallas import tpu as pltpu
from jax.experimental.pallas import tpu_sc as plsc

NC = 2
NS = 16
L = 128


def _stage_a(x, W1, b1, Wg, rb):
    n, d = x.shape
    h = W1.shape[1]

    def body(x_ref, w1_ref, b1_ref, wg_ref, out_ref):
        h1 = jnp.maximum(
            jnp.dot(x_ref[...], w1_ref[...],
                    preferred_element_type=jnp.float32) + b1_ref[...][None, :],
            0.0)
        out_ref[...] = jnp.dot(h1, wg_ref[...],
                               preferred_element_type=jnp.float32)

    return pl.pallas_call(
        body,
        grid=(n // rb,),
        in_specs=[
            pl.BlockSpec((rb, d), lambda i: (i, 0)),
            pl.BlockSpec((d, h), lambda i: (0, 0)),
            pl.BlockSpec((h,), lambda i: (0,)),
            pl.BlockSpec((h, h), lambda i: (0, 0)),
        ],
        out_specs=pl.BlockSpec((rb, h), lambda i: (i, 0)),
        out_shape=jax.ShapeDtypeStruct((n, h), jnp.float32),
    )(x, W1, b1, Wg)


def _stage_b(hw, c0, c1, rb):
    n, h = hw.shape
    hh = h // 2

    def body(hw_ref, c0_ref, c1_ref, hws0_ref, hws1_ref, dinv_ref):
        deg = c0_ref[...] + c1_ref[...] + 1.0
        dinv = lax.rsqrt(deg)
        hws = hw_ref[...] * dinv
        hws0_ref[...] = hws[:, :hh]
        hws1_ref[...] = hws[:, hh:]
        dinv_ref[...] = dinv

    return pl.pallas_call(
        body,
        grid=(n // rb,),
        in_specs=[
            pl.BlockSpec((rb, h), lambda i: (i, 0)),
            pl.BlockSpec((rb, 1), lambda i: (i, 0)),
            pl.BlockSpec((rb, 1), lambda i: (i, 0)),
        ],
        out_specs=(
            pl.BlockSpec((rb, hh), lambda i: (i, 0)),
            pl.BlockSpec((rb, hh), lambda i: (i, 0)),
            pl.BlockSpec((rb, 1), lambda i: (i, 0)),
        ),
        out_shape=(
            jax.ShapeDtypeStruct((n, hh), jnp.float32),
            jax.ShapeDtypeStruct((n, hh), jnp.float32),
            jax.ShapeDtypeStruct((n, 1), jnp.float32),
        ),
    )(hw, c0, c1)


def _stage_c(agg0, agg1, hw, dinv, bg, W2, b2, W3, b3, rb):
    n, h = hw.shape
    h2 = W2.shape[1]
    o = W3.shape[1]
    hh = h // 2

    def body(a0_ref, a1_ref, hw_ref, dv_ref, bg_ref, w2_ref, b2_ref,
             w3_ref, b3_ref, out_ref):
        agg_pre = jnp.concatenate([a0_ref[...], a1_ref[...]], axis=1)
        dv = dv_ref[...]
        hnode = jnp.maximum(
            dv * agg_pre + (dv * dv) * hw_ref[...] + bg_ref[...][None, :], 0.0)
        t = jnp.maximum(
            jnp.dot(hnode, w2_ref[...],
                    preferred_element_type=jnp.float32) + b2_ref[...][None, :],
            0.0)
        out_ref[...] = jnp.dot(
            t, w3_ref[...], preferred_element_type=jnp.float32) \
            + b3_ref[...][None, :]

    return pl.pallas_call(
        body,
        grid=(n // rb,),
        in_specs=[
            pl.BlockSpec((rb, hh), lambda i: (i, 0)),
            pl.BlockSpec((rb, hh), lambda i: (i, 0)),
            pl.BlockSpec((rb, h), lambda i: (i, 0)),
            pl.BlockSpec((rb, 1), lambda i: (i, 0)),
            pl.BlockSpec((h,), lambda i: (0,)),
            pl.BlockSpec((h, h2), lambda i: (0, 0)),
            pl.BlockSpec((h2,), lambda i: (0,)),
            pl.BlockSpec((h2, o), lambda i: (0, 0)),
            pl.BlockSpec((o,), lambda i: (0,)),
        ],
        out_specs=pl.BlockSpec((rb, o), lambda i: (i, 0)),
        out_shape=jax.ShapeDtypeStruct((n, o), jnp.float32),
    )(agg0, agg1, hw, dinv, bg, W2, b2, W3, b3)


def _make_deg_kernel(n_rows, n_pad):
    rpw = n_rows // (NC * NS)
    mesh = plsc.VectorSubcoreMesh(core_axis_name="c", subcore_axis_name="s")

    @functools.partial(
        pl.kernel,
        out_type=jax.ShapeDtypeStruct((NC, n_pad), jnp.float32),
        mesh=mesh,
        scratch_types=[
            pltpu.VMEM((rpw, L), jnp.int32),
            pltpu.VMEM((L,), jnp.float32),
            pltpu.VMEM_SHARED((n_pad,), jnp.float32),
            pltpu.SemaphoreType.DMA,
        ],
    )
    def deg_kernel(dst2_hbm, ones_hbm, zeros1_hbm, out_hbm,
                   idx_v, ones_v, hist_sh, sem):
        c = lax.axis_index("c")
        s = lax.axis_index("s")
        w = c * NS + s

        @pl.when(s == 0)
        def _():
            pltpu.sync_copy(zeros1_hbm, hist_sh)
        pltpu.sync_copy(ones_hbm, ones_v)
        pltpu.sync_copy(dst2_hbm.at[pl.ds(w * rpw, rpw)], idx_v)
        plsc.subcore_barrier()

        descs = []
        for t in range(rpw):
            descs.append(pltpu.async_copy(
                ones_v, hist_sh.at[idx_v.at[t]], sem, add=True))
        for d in descs:
            d.wait()

        plsc.subcore_barrier()

        @pl.when(s == 0)
        def _():
            pltpu.sync_copy(hist_sh, out_hbm.at[c])

    return deg_kernel


def _make_agg_kernel(n_rows, n, n_pad, hh):
    rpw = n_rows // NS
    ib = 16
    assert rpw % ib == 0
    wo_w = NS
    while n % wo_w != 0 or (n // wo_w) % 8 != 0:
        wo_w -= 1
    nps = n // wo_w
    mesh = plsc.VectorSubcoreMesh(core_axis_name="c", subcore_axis_name="s")

    @functools.partial(
        pl.kernel,
        out_type=(jax.ShapeDtypeStruct((n, hh), jnp.float32),
                  jax.ShapeDtypeStruct((n, hh), jnp.float32)),
        mesh=mesh,
        scratch_types=[
            pltpu.VMEM((2, ib, L), jnp.int32),
            pltpu.VMEM((2, ib, L), jnp.int32),
            pltpu.VMEM((L, hh), jnp.float32),
            pltpu.VMEM((L, hh), jnp.float32),
            pltpu.VMEM_SHARED((n_pad, hh), jnp.float32),
            pltpu.SemaphoreType.DMA,
            pltpu.SemaphoreType.DMA,
        ],
    )
    def agg_kernel(hws0_hbm, hws1_hbm, src2_hbm, dst2_hbm, zeros2_hbm,
                   out0_hbm, out1_hbm,
                   idxs_v, idxd_v, rows_a, rows_b, agg_sh, gsem, ssem):
        c = lax.axis_index("c")
        s = lax.axis_index("s")

        @pl.when(s == 0)
        def _():
            pltpu.sync_copy(zeros2_hbm, agg_sh)
        plsc.subcore_barrier()

        def run(tbl_hbm):
            bufs = (rows_a, rows_b)
            base = s * rpw
            gd = [None] * rpw
            sd = [None] * rpw

            def scatter(j):
                blk, off = divmod(j, ib)
                gd[j].wait()
                sd[j] = pltpu.async_copy(
                    bufs[j % 2], agg_sh.at[idxd_v.at[blk % 2].at[off]],
                    ssem, add=True)

            for j in range(rpw):
                blk, off = divmod(j, ib)
                p = blk % 2
                if off == 0:
                    pltpu.sync_copy(
                        src2_hbm.at[pl.ds(base + blk * ib, ib)], idxs_v.at[p])
                    pltpu.sync_copy(
                        dst2_hbm.at[pl.ds(base + blk * ib, ib)], idxd_v.at[p])
                if j >= 2:
                    sd[j - 2].wait()
                gd[j] = pltpu.async_copy(
                    tbl_hbm.at[idxs_v.at[p].at[off]], bufs[j % 2], gsem)
                if j >= 1:
                    scatter(j - 1)
            scatter(rpw - 1)
            sd[rpw - 2].wait()
            sd[rpw - 1].wait()

        @pl.when(c == 0)
        def _():
            run(hws0_hbm)

        @pl.when(c == 1)
        def _():
            run(hws1_hbm)

        plsc.subcore_barrier()

        @pl.when(jnp.logical_and(c == 0, s < wo_w))
        def _():
            pltpu.sync_copy(agg_sh.at[pl.ds(s * nps, nps)],
                            out0_hbm.at[pl.ds(s * nps, nps)])

        @pl.when(jnp.logical_and(c == 1, s < wo_w))
        def _():
            pltpu.sync_copy(agg_sh.at[pl.ds(s * nps, nps)],
                            out1_hbm.at[pl.ds(s * nps, nps)])

    return agg_kernel


def _make_pair_kernel(n_idx_rows, p, o, left_rows):
    nw = NC * NS
    nt = -(-(-(-n_idx_rows // nw)) // 8) * 8
    tail_valid = p - (left_rows - 1) * L
    depth = 4
    mesh = plsc.VectorSubcoreMesh(core_axis_name="c", subcore_axis_name="s")

    @functools.partial(
        pl.kernel,
        out_type=jax.ShapeDtypeStruct((2 * p, o), jnp.float32),
        mesh=mesh,
        scratch_types=[
            pltpu.VMEM((nt, L), jnp.int32),
            [pltpu.VMEM((L, o), jnp.float32) for _ in range(depth)],
            pltpu.SemaphoreType.DMA,
            pltpu.SemaphoreType.DMA,
        ],
    )
    def pair_kernel(etab_hbm, idx2_hbm, out_hbm, idx_v, bufs, gsem, wsem):
        c = lax.axis_index("c")
        s = lax.axis_index("s")
        w = c * NS + s
        k0 = w * nt

        pltpu.sync_copy(idx2_hbm.at[pl.ds(k0, nt)], idx_v)

        gd = [None] * nt

        def write_out(t):
            k = k0 + t
            is_tail = jnp.logical_or(k == left_rows - 1,
                                     k == 2 * left_rows - 1)
            gd[t].wait()

            def obase():
                return pl.multiple_of(
                    jnp.where(k < left_rows, k * L,
                              p + (k - left_rows) * L), 8)

            @pl.when(is_tail)
            def _():
                pltpu.sync_copy(bufs[t % depth].at[pl.ds(0, tail_valid)],
                                out_hbm.at[pl.ds(obase(), tail_valid)])

            @pl.when(jnp.logical_not(is_tail))
            def _():
                pltpu.async_copy(bufs[t % depth],
                                 out_hbm.at[pl.ds(obase(), L)], wsem)

        def drain(t):
            k = k0 + t
            pred = jnp.logical_and(
                k < n_idx_rows,
                jnp.logical_not(jnp.logical_or(k == left_rows - 1,
                                               k == 2 * left_rows - 1)))

            @pl.when(pred)
            def _():
                pltpu.make_async_copy(bufs[t % depth],
                                      out_hbm.at[pl.ds(0, L)], wsem).wait()

        for t in range(nt):
            live = k0 + t < n_idx_rows

            if t >= depth:
                drain(t - depth)

            @pl.when(live)
            def _():
                gd[t] = pltpu.async_copy(
                    etab_hbm.at[idx_v.at[t]], bufs[t % depth], gsem)

            if t >= 1:
                tm = t - 1

                @pl.when(k0 + tm < n_idx_rows)
                def _():
                    write_out(tm)

        @pl.when(k0 + nt - 1 < n_idx_rows)
        def _():
            write_out(nt - 1)
        for t in range(max(0, nt - depth), nt):
            drain(t)

    return pair_kernel


def kernel(x, edge_index, pair_idxs_left, pair_idxs_right, y,
           W1, b1, Wg, bg, W2, b2, W3, b3):
    n, d = x.shape
    e = edge_index.shape[1]
    p = pair_idxs_left.shape[0]
    h = W1.shape[1]
    o = W3.shape[1]
    hh = h // 2
    rb = 1000 if n % 1000 == 0 else 8

    unit = NC * NS * L
    e_pad = -(-e // unit) * unit
    src_p = jnp.concatenate(
        [edge_index[0], jnp.zeros((e_pad - e,), jnp.int32)]).reshape(-1, L)
    dst_p = jnp.concatenate(
        [edge_index[1], jnp.full((e_pad - e,), n, jnp.int32)]).reshape(-1, L)
    n_rows = e_pad // L
    n_pad1 = -(-(n + 1) // L) * L
    n_pad2 = n + 8

    ones_l = jnp.ones((L,), jnp.float32)
    zeros1 = jnp.zeros((n_pad1,), jnp.float32)
    zeros2 = jnp.zeros((n_pad2, hh), jnp.float32)

    counts = _make_deg_kernel(n_rows, n_pad1)(dst_p, ones_l, zeros1)
    hw = _stage_a(x, W1, b1, Wg, rb)
    c0 = counts[0, :n][:, None]
    c1 = counts[1, :n][:, None]
    hws0, hws1, dinv = _stage_b(hw, c0, c1, rb)
    agg0, agg1 = _make_agg_kernel(n_rows, n, n_pad2, hh)(
        hws0, hws1, src_p, dst_p, zeros2)
    etab = _stage_c(agg0, agg1, hw, dinv, bg, W2, b2, W3, b3, rb)

    left_rows = -(-p // L)
    ipad = left_rows * L - p
    zpad = jnp.zeros((ipad,), jnp.int32)
    n_idx_rows = 2 * left_rows
    nw = NC * NS
    rows_tot = -(-(-(-n_idx_rows // nw)) // 8) * 8 * nw
    zpad2 = jnp.zeros(((rows_tot - n_idx_rows) * L,), jnp.int32)
    idx_all = jnp.concatenate(
        [pair_idxs_left, zpad, pair_idxs_right, zpad, zpad2]).reshape(-1, L)
    flat = _make_pair_kernel(n_idx_rows, p, o, left_rows)(etab, idx_all)
    return flat.reshape(2, p, o), y


# speedup vs baseline: 9.4470x; 1.0001x over previous
import functools

import jax
import jax.numpy as jnp
from jax import lax
from jax.experimental import pallas as pl
from jax.experimental.pallas import tpu as pltpu
from jax.experimental.pallas import tpu_sc as plsc

NC = 2
NS = 16
L = 128


def _stage_a(x, W1, b1, Wg, rb):
    n, d = x.shape
    h = W1.shape[1]

    def body(x_ref, w1_ref, b1_ref, wg_ref, out_ref):
        h1 = jnp.maximum(
            jnp.dot(x_ref[...], w1_ref[...],
                    preferred_element_type=jnp.float32) + b1_ref[...][None, :],
            0.0)
        out_ref[...] = jnp.dot(h1, wg_ref[...],
                               preferred_element_type=jnp.float32)

    return pl.pallas_call(
        body,
        grid=(n // rb,),
        in_specs=[
            pl.BlockSpec((rb, d), lambda i: (i, 0)),
            pl.BlockSpec((d, h), lambda i: (0, 0)),
            pl.BlockSpec((h,), lambda i: (0,)),
            pl.BlockSpec((h, h), lambda i: (0, 0)),
        ],
        out_specs=pl.BlockSpec((rb, h), lambda i: (i, 0)),
        out_shape=jax.ShapeDtypeStruct((n, h), jnp.float32),
    )(x, W1, b1, Wg)


def _stage_b(hw, c0, c1, rb):
    n, h = hw.shape
    hh = h // 2

    def body(hw_ref, c0_ref, c1_ref, hws0_ref, hws1_ref, dinv_ref):
        deg = c0_ref[...] + c1_ref[...] + 1.0
        dinv = lax.rsqrt(deg)
        hws = hw_ref[...] * dinv
        hws0_ref[...] = hws[:, :hh]
        hws1_ref[...] = hws[:, hh:]
        dinv_ref[...] = dinv

    return pl.pallas_call(
        body,
        grid=(n // rb,),
        in_specs=[
            pl.BlockSpec((rb, h), lambda i: (i, 0)),
            pl.BlockSpec((rb, 1), lambda i: (i, 0)),
            pl.BlockSpec((rb, 1), lambda i: (i, 0)),
        ],
        out_specs=(
            pl.BlockSpec((rb, hh), lambda i: (i, 0)),
            pl.BlockSpec((rb, hh), lambda i: (i, 0)),
            pl.BlockSpec((rb, 1), lambda i: (i, 0)),
        ),
        out_shape=(
            jax.ShapeDtypeStruct((n, hh), jnp.float32),
            jax.ShapeDtypeStruct((n, hh), jnp.float32),
            jax.ShapeDtypeStruct((n, 1), jnp.float32),
        ),
    )(hw, c0, c1)


def _stage_c(agg0, agg1, hw, dinv, bg, W2, b2, W3, b3, rb):
    n, h = hw.shape
    h2 = W2.shape[1]
    o = W3.shape[1]
    hh = h // 2

    def body(a0_ref, a1_ref, hw_ref, dv_ref, bg_ref, w2_ref, b2_ref,
             w3_ref, b3_ref, out_ref):
        agg_pre = jnp.concatenate([a0_ref[...], a1_ref[...]], axis=1)
        dv = dv_ref[...]
        hnode = jnp.maximum(
            dv * agg_pre + (dv * dv) * hw_ref[...] + bg_ref[...][None, :], 0.0)
        t = jnp.maximum(
            jnp.dot(hnode, w2_ref[...],
                    preferred_element_type=jnp.float32) + b2_ref[...][None, :],
            0.0)
        out_ref[...] = jnp.dot(
            t, w3_ref[...], preferred_element_type=jnp.float32) \
            + b3_ref[...][None, :]

    return pl.pallas_call(
        body,
        grid=(n // rb,),
        in_specs=[
            pl.BlockSpec((rb, hh), lambda i: (i, 0)),
            pl.BlockSpec((rb, hh), lambda i: (i, 0)),
            pl.BlockSpec((rb, h), lambda i: (i, 0)),
            pl.BlockSpec((rb, 1), lambda i: (i, 0)),
            pl.BlockSpec((h,), lambda i: (0,)),
            pl.BlockSpec((h, h2), lambda i: (0, 0)),
            pl.BlockSpec((h2,), lambda i: (0,)),
            pl.BlockSpec((h2, o), lambda i: (0, 0)),
            pl.BlockSpec((o,), lambda i: (0,)),
        ],
        out_specs=pl.BlockSpec((rb, o), lambda i: (i, 0)),
        out_shape=jax.ShapeDtypeStruct((n, o), jnp.float32),
    )(agg0, agg1, hw, dinv, bg, W2, b2, W3, b3)


def _make_deg_kernel(n_rows, n_pad):
    rpw = n_rows // (NC * NS)
    mesh = plsc.VectorSubcoreMesh(core_axis_name="c", subcore_axis_name="s")

    @functools.partial(
        pl.kernel,
        out_type=jax.ShapeDtypeStruct((NC, n_pad), jnp.float32),
        mesh=mesh,
        scratch_types=[
            pltpu.VMEM((rpw, L), jnp.int32),
            pltpu.VMEM((L,), jnp.float32),
            pltpu.VMEM_SHARED((n_pad,), jnp.float32),
            pltpu.SemaphoreType.DMA,
        ],
    )
    def deg_kernel(dst2_hbm, ones_hbm, zeros1_hbm, out_hbm,
                   idx_v, ones_v, hist_sh, sem):
        c = lax.axis_index("c")
        s = lax.axis_index("s")
        w = c * NS + s

        @pl.when(s == 0)
        def _():
            pltpu.sync_copy(zeros1_hbm, hist_sh)
        pltpu.sync_copy(ones_hbm, ones_v)
        pltpu.sync_copy(dst2_hbm.at[pl.ds(w * rpw, rpw)], idx_v)
        plsc.subcore_barrier()

        descs = []
        for t in range(rpw):
            descs.append(pltpu.async_copy(
                ones_v, hist_sh.at[idx_v.at[t]], sem, add=True))
        for d in descs:
            d.wait()

        plsc.subcore_barrier()

        @pl.when(s == 0)
        def _():
            pltpu.sync_copy(hist_sh, out_hbm.at[c])

    return deg_kernel


def _make_agg_kernel(n_rows, n, n_pad, hh):
    rpw = n_rows // NS
    ib = 16
    assert rpw % ib == 0
    wo_w = NS
    while n % wo_w != 0 or (n // wo_w) % 8 != 0:
        wo_w -= 1
    nps = n // wo_w
    mesh = plsc.VectorSubcoreMesh(core_axis_name="c", subcore_axis_name="s")

    @functools.partial(
        pl.kernel,
        out_type=(jax.ShapeDtypeStruct((n, hh), jnp.float32),
                  jax.ShapeDtypeStruct((n, hh), jnp.float32)),
        mesh=mesh,
        scratch_types=[
            pltpu.VMEM((2, ib, L), jnp.int32),
            pltpu.VMEM((2, ib, L), jnp.int32),
            pltpu.VMEM((L, hh), jnp.float32),
            pltpu.VMEM((L, hh), jnp.float32),
            pltpu.VMEM_SHARED((n_pad, hh), jnp.float32),
            pltpu.SemaphoreType.DMA,
            pltpu.SemaphoreType.DMA,
        ],
    )
    def agg_kernel(hws0_hbm, hws1_hbm, src2_hbm, dst2_hbm, zeros2_hbm,
                   out0_hbm, out1_hbm,
                   idxs_v, idxd_v, rows_a, rows_b, agg_sh, gsem, ssem):
        c = lax.axis_index("c")
        s = lax.axis_index("s")

        @pl.when(s < wo_w)
        def _():
            pltpu.sync_copy(zeros2_hbm.at[pl.ds(s * nps, nps)],
                            agg_sh.at[pl.ds(s * nps, nps)])

        @pl.when(s == 0)
        def _():
            pltpu.sync_copy(zeros2_hbm.at[pl.ds(n, n_pad - n)],
                            agg_sh.at[pl.ds(n, n_pad - n)])
        plsc.subcore_barrier()

        def run(tbl_hbm):
            bufs = (rows_a, rows_b)
            base = s * rpw
            gd = [None] * rpw
            sd = [None] * rpw

            def scatter(j):
                blk, off = divmod(j, ib)
                gd[j].wait()
                sd[j] = pltpu.async_copy(
                    bufs[j % 2], agg_sh.at[idxd_v.at[blk % 2].at[off]],
                    ssem, add=True)

            for j in range(rpw):
                blk, off = divmod(j, ib)
                p = blk % 2
                if off == 0:
                    pltpu.sync_copy(
                        src2_hbm.at[pl.ds(base + blk * ib, ib)], idxs_v.at[p])
                    pltpu.sync_copy(
                        dst2_hbm.at[pl.ds(base + blk * ib, ib)], idxd_v.at[p])
                if j >= 2:
                    sd[j - 2].wait()
                gd[j] = pltpu.async_copy(
                    tbl_hbm.at[idxs_v.at[p].at[off]], bufs[j % 2], gsem)
                if j >= 1:
                    scatter(j - 1)
            scatter(rpw - 1)
            sd[rpw - 2].wait()
            sd[rpw - 1].wait()

        @pl.when(c == 0)
        def _():
            run(hws0_hbm)

        @pl.when(c == 1)
        def _():
            run(hws1_hbm)

        plsc.subcore_barrier()

        @pl.when(jnp.logical_and(c == 0, s < wo_w))
        def _():
            pltpu.sync_copy(agg_sh.at[pl.ds(s * nps, nps)],
                            out0_hbm.at[pl.ds(s * nps, nps)])

        @pl.when(jnp.logical_and(c == 1, s < wo_w))
        def _():
            pltpu.sync_copy(agg_sh.at[pl.ds(s * nps, nps)],
                            out1_hbm.at[pl.ds(s * nps, nps)])

    return agg_kernel


def _make_pair_kernel(n_idx_rows, p, o, left_rows):
    nw = NC * NS
    nt = -(-(-(-n_idx_rows // nw)) // 8) * 8
    tail_valid = p - (left_rows - 1) * L
    depth = 4
    mesh = plsc.VectorSubcoreMesh(core_axis_name="c", subcore_axis_name="s")

    @functools.partial(
        pl.kernel,
        out_type=jax.ShapeDtypeStruct((2 * p, o), jnp.float32),
        mesh=mesh,
        scratch_types=[
            pltpu.VMEM((nt, L), jnp.int32),
            [pltpu.VMEM((L, o), jnp.float32) for _ in range(depth)],
            pltpu.SemaphoreType.DMA,
            pltpu.SemaphoreType.DMA,
        ],
    )
    def pair_kernel(etab_hbm, idx2_hbm, out_hbm, idx_v, bufs, gsem, wsem):
        c = lax.axis_index("c")
        s = lax.axis_index("s")
        w = c * NS + s
        k0 = w * nt

        pltpu.sync_copy(idx2_hbm.at[pl.ds(k0, nt)], idx_v)

        gd = [None] * nt

        def write_out(t):
            k = k0 + t
            is_tail = jnp.logical_or(k == left_rows - 1,
                                     k == 2 * left_rows - 1)
            gd[t].wait()

            def obase():
                return pl.multiple_of(
                    jnp.where(k < left_rows, k * L,
                              p + (k - left_rows) * L), 8)

            @pl.when(is_tail)
            def _():
                pltpu.sync_copy(bufs[t % depth].at[pl.ds(0, tail_valid)],
                                out_hbm.at[pl.ds(obase(), tail_valid)])

            @pl.when(jnp.logical_not(is_tail))
            def _():
                pltpu.async_copy(bufs[t % depth],
                                 out_hbm.at[pl.ds(obase(), L)], wsem)

        def drain(t):
            k = k0 + t
            pred = jnp.logical_and(
                k < n_idx_rows,
                jnp.logical_not(jnp.logical_or(k == left_rows - 1,
                                               k == 2 * left_rows - 1)))

            @pl.when(pred)
            def _():
                pltpu.make_async_copy(bufs[t % depth],
                                      out_hbm.at[pl.ds(0, L)], wsem).wait()

        for t in range(nt):
            live = k0 + t < n_idx_rows

            if t >= depth:
                drain(t - depth)

            @pl.when(live)
            def _():
                gd[t] = pltpu.async_copy(
                    etab_hbm.at[idx_v.at[t]], bufs[t % depth], gsem)

            if t >= 1:
                tm = t - 1

                @pl.when(k0 + tm < n_idx_rows)
                def _():
                    write_out(tm)

        @pl.when(k0 + nt - 1 < n_idx_rows)
        def _():
            write_out(nt - 1)
        for t in range(max(0, nt - depth), nt):
            drain(t)

    return pair_kernel


def kernel(x, edge_index, pair_idxs_left, pair_idxs_right, y,
           W1, b1, Wg, bg, W2, b2, W3, b3):
    n, d = x.shape
    e = edge_index.shape[1]
    p = pair_idxs_left.shape[0]
    h = W1.shape[1]
    o = W3.shape[1]
    hh = h // 2
    rb = 1000 if n % 1000 == 0 else 8

    unit = NC * NS * L
    e_pad = -(-e // unit) * unit
    src_p = jnp.concatenate(
        [edge_index[0], jnp.zeros((e_pad - e,), jnp.int32)]).reshape(-1, L)
    dst_p = jnp.concatenate(
        [edge_index[1], jnp.full((e_pad - e,), n, jnp.int32)]).reshape(-1, L)
    n_rows = e_pad // L
    n_pad1 = -(-(n + 1) // L) * L
    n_pad2 = n + 8

    ones_l = jnp.ones((L,), jnp.float32)
    zeros1 = jnp.zeros((n_pad1,), jnp.float32)
    zeros2 = jnp.zeros((n_pad2, hh), jnp.float32)

    counts = _make_deg_kernel(n_rows, n_pad1)(dst_p, ones_l, zeros1)
    hw = _stage_a(x, W1, b1, Wg, rb)
    c0 = counts[0, :n][:, None]
    c1 = counts[1, :n][:, None]
    hws0, hws1, dinv = _stage_b(hw, c0, c1, rb)
    agg0, agg1 = _make_agg_kernel(n_rows, n, n_pad2, hh)(
        hws0, hws1, src_p, dst_p, zeros2)
    etab = _stage_c(agg0, agg1, hw, dinv, bg, W2, b2, W3, b3, rb)

    left_rows = -(-p // L)
    ipad = left_rows * L - p
    zpad = jnp.zeros((ipad,), jnp.int32)
    n_idx_rows = 2 * left_rows
    nw = NC * NS
    rows_tot = -(-(-(-n_idx_rows // nw)) // 8) * 8 * nw
    zpad2 = jnp.zeros(((rows_tot - n_idx_rows) * L,), jnp.int32)
    idx_all = jnp.concatenate(
        [pair_idxs_left, zpad, pair_idxs_right, zpad, zpad2]).reshape(-1, L)
    flat = _make_pair_kernel(n_idx_rows, p, o, left_rows)(etab, idx_all)
    return flat.reshape(2, p, o), y


# speedup vs baseline: 9.4913x; 1.0047x over previous
import functools

import jax
import jax.numpy as jnp
from jax import lax
from jax.experimental import pallas as pl
from jax.experimental.pallas import tpu as pltpu
from jax.experimental.pallas import tpu_sc as plsc

NC = 2
NS = 16
L = 128


def _stage_a(x, W1, b1, Wg, rb):
    n, d = x.shape
    h = W1.shape[1]

    def body(x_ref, w1_ref, b1_ref, wg_ref, out_ref):
        h1 = jnp.maximum(
            jnp.dot(x_ref[...], w1_ref[...],
                    preferred_element_type=jnp.float32) + b1_ref[...][None, :],
            0.0)
        out_ref[...] = jnp.dot(h1, wg_ref[...],
                               preferred_element_type=jnp.float32)

    return pl.pallas_call(
        body,
        grid=(n // rb,),
        in_specs=[
            pl.BlockSpec((rb, d), lambda i: (i, 0)),
            pl.BlockSpec((d, h), lambda i: (0, 0)),
            pl.BlockSpec((h,), lambda i: (0,)),
            pl.BlockSpec((h, h), lambda i: (0, 0)),
        ],
        out_specs=pl.BlockSpec((rb, h), lambda i: (i, 0)),
        out_shape=jax.ShapeDtypeStruct((n, h), jnp.float32),
    )(x, W1, b1, Wg)


def _stage_b(hw, c0, c1, rb):
    n, h = hw.shape
    hh = h // 2

    def body(hw_ref, c0_ref, c1_ref, hws0_ref, hws1_ref, dinv_ref):
        deg = c0_ref[...] + c1_ref[...] + 1.0
        dinv = lax.rsqrt(deg)
        hws = hw_ref[...] * dinv
        hws0_ref[...] = hws[:, :hh]
        hws1_ref[...] = hws[:, hh:]
        dinv_ref[...] = dinv

    return pl.pallas_call(
        body,
        grid=(n // rb,),
        in_specs=[
            pl.BlockSpec((rb, h), lambda i: (i, 0)),
            pl.BlockSpec((rb, 1), lambda i: (i, 0)),
            pl.BlockSpec((rb, 1), lambda i: (i, 0)),
        ],
        out_specs=(
            pl.BlockSpec((rb, hh), lambda i: (i, 0)),
            pl.BlockSpec((rb, hh), lambda i: (i, 0)),
            pl.BlockSpec((rb, 1), lambda i: (i, 0)),
        ),
        out_shape=(
            jax.ShapeDtypeStruct((n, hh), jnp.float32),
            jax.ShapeDtypeStruct((n, hh), jnp.float32),
            jax.ShapeDtypeStruct((n, 1), jnp.float32),
        ),
    )(hw, c0, c1)


def _stage_c(agg0, agg1, hw, dinv, bg, W2, b2, W3, b3, rb):
    n, h = hw.shape
    h2 = W2.shape[1]
    o = W3.shape[1]
    hh = h // 2

    def body(a0_ref, a1_ref, hw_ref, dv_ref, bg_ref, w2_ref, b2_ref,
             w3_ref, b3_ref, out_ref):
        agg_pre = jnp.concatenate([a0_ref[...], a1_ref[...]], axis=1)
        dv = dv_ref[...]
        hnode = jnp.maximum(
            dv * agg_pre + (dv * dv) * hw_ref[...] + bg_ref[...][None, :], 0.0)
        t = jnp.maximum(
            jnp.dot(hnode, w2_ref[...],
                    preferred_element_type=jnp.float32) + b2_ref[...][None, :],
            0.0)
        out_ref[...] = jnp.dot(
            t, w3_ref[...], preferred_element_type=jnp.float32) \
            + b3_ref[...][None, :]

    return pl.pallas_call(
        body,
        grid=(n // rb,),
        in_specs=[
            pl.BlockSpec((rb, hh), lambda i: (i, 0)),
            pl.BlockSpec((rb, hh), lambda i: (i, 0)),
            pl.BlockSpec((rb, h), lambda i: (i, 0)),
            pl.BlockSpec((rb, 1), lambda i: (i, 0)),
            pl.BlockSpec((h,), lambda i: (0,)),
            pl.BlockSpec((h, h2), lambda i: (0, 0)),
            pl.BlockSpec((h2,), lambda i: (0,)),
            pl.BlockSpec((h2, o), lambda i: (0, 0)),
            pl.BlockSpec((o,), lambda i: (0,)),
        ],
        out_specs=pl.BlockSpec((rb, o), lambda i: (i, 0)),
        out_shape=jax.ShapeDtypeStruct((n, o), jnp.float32),
    )(agg0, agg1, hw, dinv, bg, W2, b2, W3, b3)


def _make_deg_kernel(n_rows, n_pad):
    rpw = n_rows // (NC * NS)
    mesh = plsc.VectorSubcoreMesh(core_axis_name="c", subcore_axis_name="s")

    @functools.partial(
        pl.kernel,
        out_type=jax.ShapeDtypeStruct((NC, n_pad), jnp.float32),
        mesh=mesh,
        scratch_types=[
            pltpu.VMEM((rpw, L), jnp.int32),
            pltpu.VMEM((L,), jnp.float32),
            pltpu.VMEM_SHARED((n_pad,), jnp.float32),
            pltpu.SemaphoreType.DMA,
        ],
    )
    def deg_kernel(dst2_hbm, ones_hbm, zeros1_hbm, out_hbm,
                   idx_v, ones_v, hist_sh, sem):
        c = lax.axis_index("c")
        s = lax.axis_index("s")
        w = c * NS + s

        @pl.when(s == 0)
        def _():
            pltpu.sync_copy(zeros1_hbm, hist_sh)
        pltpu.sync_copy(ones_hbm, ones_v)
        pltpu.sync_copy(dst2_hbm.at[pl.ds(w * rpw, rpw)], idx_v)
        plsc.subcore_barrier()

        descs = []
        for t in range(rpw):
            descs.append(pltpu.async_copy(
                ones_v, hist_sh.at[idx_v.at[t]], sem, add=True))
        for d in descs:
            d.wait()

        plsc.subcore_barrier()

        @pl.when(s == 0)
        def _():
            pltpu.sync_copy(hist_sh, out_hbm.at[c])

    return deg_kernel


def _make_agg_kernel(n_rows, n, n_pad, hh):
    rpw = n_rows // NS
    ib = 8
    nbuf = 4
    assert rpw % ib == 0
    nblk = rpw // ib
    wo_w = NS
    while n % wo_w != 0 or (n // wo_w) % 8 != 0:
        wo_w -= 1
    nps = n // wo_w
    mesh = plsc.VectorSubcoreMesh(core_axis_name="c", subcore_axis_name="s")

    @functools.partial(
        pl.kernel,
        out_type=(jax.ShapeDtypeStruct((n, hh), jnp.float32),
                  jax.ShapeDtypeStruct((n, hh), jnp.float32)),
        mesh=mesh,
        scratch_types=[
            pltpu.VMEM((nbuf, ib, L), jnp.int32),
            pltpu.VMEM((nbuf, ib, L), jnp.int32),
            pltpu.VMEM((L, hh), jnp.float32),
            pltpu.VMEM((L, hh), jnp.float32),
            pltpu.VMEM_SHARED((n_pad, hh), jnp.float32),
            pltpu.SemaphoreType.DMA,
            pltpu.SemaphoreType.DMA,
            pltpu.SemaphoreType.DMA,
            pltpu.SemaphoreType.DMA,
        ],
    )
    def agg_kernel(hws0_hbm, hws1_hbm, src2_hbm, dst2_hbm, zeros2_hbm,
                   out0_hbm, out1_hbm,
                   idxs_v, idxd_v, rows_a, rows_b, agg_sh,
                   gsem, ssem, isem, zsem):
        c = lax.axis_index("c")
        s = lax.axis_index("s")

        @pl.when(s < wo_w)
        def _():
            pltpu.async_copy(zeros2_hbm.at[pl.ds(s * nps, nps)],
                             agg_sh.at[pl.ds(s * nps, nps)], zsem)

        @pl.when(s == 0)
        def _():
            pltpu.async_copy(zeros2_hbm.at[pl.ds(n, n_pad - n)],
                             agg_sh.at[pl.ds(n, n_pad - n)], zsem)

        def run(tbl_hbm):
            bufs = (rows_a, rows_b)
            base = s * rpw
            gd = [None] * rpw
            sd = [None] * rpw
            ld = [None] * nblk

            def iload(b):
                ld[b] = (
                    pltpu.async_copy(
                        src2_hbm.at[pl.ds(base + b * ib, ib)],
                        idxs_v.at[b % nbuf], isem),
                    pltpu.async_copy(
                        dst2_hbm.at[pl.ds(base + b * ib, ib)],
                        idxd_v.at[b % nbuf], isem),
                )

            iload(0)

            @pl.when(s < wo_w)
            def _():
                pltpu.make_async_copy(
                    zeros2_hbm.at[pl.ds(s * nps, nps)],
                    agg_sh.at[pl.ds(s * nps, nps)], zsem).wait()

            @pl.when(s == 0)
            def _():
                pltpu.make_async_copy(
                    zeros2_hbm.at[pl.ds(n, n_pad - n)],
                    agg_sh.at[pl.ds(n, n_pad - n)], zsem).wait()
            plsc.subcore_barrier()

            def scatter(j):
                blk, off = divmod(j, ib)
                gd[j].wait()
                sd[j] = pltpu.async_copy(
                    bufs[j % 2], agg_sh.at[idxd_v.at[blk % nbuf].at[off]],
                    ssem, add=True)

            for j in range(rpw):
                blk, off = divmod(j, ib)
                p = blk % nbuf
                if off == 0:
                    for d in ld[blk]:
                        d.wait()
                    if blk + 1 < nblk:
                        iload(blk + 1)
                if j >= 2:
                    sd[j - 2].wait()
                gd[j] = pltpu.async_copy(
                    tbl_hbm.at[idxs_v.at[p].at[off]], bufs[j % 2], gsem)
                if j >= 1:
                    scatter(j - 1)
            scatter(rpw - 1)
            sd[rpw - 2].wait()
            sd[rpw - 1].wait()

        @pl.when(c == 0)
        def _():
            run(hws0_hbm)

        @pl.when(c == 1)
        def _():
            run(hws1_hbm)

        plsc.subcore_barrier()

        @pl.when(jnp.logical_and(c == 0, s < wo_w))
        def _():
            pltpu.sync_copy(agg_sh.at[pl.ds(s * nps, nps)],
                            out0_hbm.at[pl.ds(s * nps, nps)])

        @pl.when(jnp.logical_and(c == 1, s < wo_w))
        def _():
            pltpu.sync_copy(agg_sh.at[pl.ds(s * nps, nps)],
                            out1_hbm.at[pl.ds(s * nps, nps)])

    return agg_kernel


def _make_pair_kernel(n_idx_rows, p, o, left_rows):
    nw = NC * NS
    nt = -(-(-(-n_idx_rows // nw)) // 8) * 8
    tail_valid = p - (left_rows - 1) * L
    depth = 4
    mesh = plsc.VectorSubcoreMesh(core_axis_name="c", subcore_axis_name="s")

    @functools.partial(
        pl.kernel,
        out_type=jax.ShapeDtypeStruct((2 * p, o), jnp.float32),
        mesh=mesh,
        scratch_types=[
            pltpu.VMEM((nt, L), jnp.int32),
            [pltpu.VMEM((L, o), jnp.float32) for _ in range(depth)],
            pltpu.SemaphoreType.DMA,
            pltpu.SemaphoreType.DMA,
        ],
    )
    def pair_kernel(etab_hbm, idx2_hbm, out_hbm, idx_v, bufs, gsem, wsem):
        c = lax.axis_index("c")
        s = lax.axis_index("s")
        w = c * NS + s
        k0 = w * nt

        pltpu.sync_copy(idx2_hbm.at[pl.ds(k0, nt)], idx_v)

        gd = [None] * nt

        def write_out(t):
            k = k0 + t
            is_tail = jnp.logical_or(k == left_rows - 1,
                                     k == 2 * left_rows - 1)
            gd[t].wait()

            def obase():
                return pl.multiple_of(
                    jnp.where(k < left_rows, k * L,
                              p + (k - left_rows) * L), 8)

            @pl.when(is_tail)
            def _():
                pltpu.sync_copy(bufs[t % depth].at[pl.ds(0, tail_valid)],
                                out_hbm.at[pl.ds(obase(), tail_valid)])

            @pl.when(jnp.logical_not(is_tail))
            def _():
                pltpu.async_copy(bufs[t % depth],
                                 out_hbm.at[pl.ds(obase(), L)], wsem)

        def drain(t):
            k = k0 + t
            pred = jnp.logical_and(
                k < n_idx_rows,
                jnp.logical_not(jnp.logical_or(k == left_rows - 1,
                                               k == 2 * left_rows - 1)))

            @pl.when(pred)
            def _():
                pltpu.make_async_copy(bufs[t % depth],
                                      out_hbm.at[pl.ds(0, L)], wsem).wait()

        for t in range(nt):
            live = k0 + t < n_idx_rows

            if t >= depth:
                drain(t - depth)

            @pl.when(live)
            def _():
                gd[t] = pltpu.async_copy(
                    etab_hbm.at[idx_v.at[t]], bufs[t % depth], gsem)

            if t >= 1:
                tm = t - 1

                @pl.when(k0 + tm < n_idx_rows)
                def _():
                    write_out(tm)

        @pl.when(k0 + nt - 1 < n_idx_rows)
        def _():
            write_out(nt - 1)
        for t in range(max(0, nt - depth), nt):
            drain(t)

    return pair_kernel


def kernel(x, edge_index, pair_idxs_left, pair_idxs_right, y,
           W1, b1, Wg, bg, W2, b2, W3, b3):
    n, d = x.shape
    e = edge_index.shape[1]
    p = pair_idxs_left.shape[0]
    h = W1.shape[1]
    o = W3.shape[1]
    hh = h // 2
    rb = 1000 if n % 1000 == 0 else 8

    unit = NC * NS * L
    e_pad = -(-e // unit) * unit
    src_p = jnp.concatenate(
        [edge_index[0], jnp.zeros((e_pad - e,), jnp.int32)]).reshape(-1, L)
    dst_p = jnp.concatenate(
        [edge_index[1], jnp.full((e_pad - e,), n, jnp.int32)]).reshape(-1, L)
    n_rows = e_pad // L
    n_pad1 = -(-(n + 1) // L) * L
    n_pad2 = n + 8

    ones_l = jnp.ones((L,), jnp.float32)
    zeros1 = jnp.zeros((n_pad1,), jnp.float32)
    zeros2 = jnp.zeros((n_pad2, hh), jnp.float32)

    counts = _make_deg_kernel(n_rows, n_pad1)(dst_p, ones_l, zeros1)
    hw = _stage_a(x, W1, b1, Wg, rb)
    c0 = counts[0, :n][:, None]
    c1 = counts[1, :n][:, None]
    hws0, hws1, dinv = _stage_b(hw, c0, c1, rb)
    agg0, agg1 = _make_agg_kernel(n_rows, n, n_pad2, hh)(
        hws0, hws1, src_p, dst_p, zeros2)
    etab = _stage_c(agg0, agg1, hw, dinv, bg, W2, b2, W3, b3, rb)

    left_rows = -(-p // L)
    ipad = left_rows * L - p
    zpad = jnp.zeros((ipad,), jnp.int32)
    n_idx_rows = 2 * left_rows
    nw = NC * NS
    rows_tot = -(-(-(-n_idx_rows // nw)) // 8) * 8 * nw
    zpad2 = jnp.zeros(((rows_tot - n_idx_rows) * L,), jnp.int32)
    idx_all = jnp.concatenate(
        [pair_idxs_left, zpad, pair_idxs_right, zpad, zpad2]).reshape(-1, L)
    flat = _make_pair_kernel(n_idx_rows, p, o, left_rows)(etab, idx_all)
    return flat.reshape(2, p, o), y
